# baseline (speedup 1.0000x reference)
import functools

import jax
import jax.numpy as jnp
from jax import lax
from jax.experimental import pallas as pl
from jax.experimental.pallas import tpu as pltpu
from jax.experimental.pallas import tpu_sc as plsc

_B, _L, _H, _D = 1024, 200, 8, 16
_VOCAB = 800532
_VPAD = 800640
_NVJ = _VPAD // 128
_NC, _NS = 2, 16
_NW = _NC * _NS

_TK = 7
_TSTEPS = 28
_TIN = _TK * 1024
_TROWS = _TK * 128

_LPW = 7

_MESH = dict(core_axis_name="c", subcore_axis_name="s")
_PARAMS = pltpu.CompilerParams(use_tc_tiling_on_sc=False,
                               needs_layout_passes=False)


def _make_table_kernel():
    mesh = plsc.VectorSubcoreMesh(**_MESH)

    @functools.partial(
        pl.kernel,
        out_type=jax.ShapeDtypeStruct((_VPAD, _D), jnp.float32),
        mesh=mesh,
        scratch_types=[
            pltpu.VMEM((2, 2, _TK, 8, 129), jnp.float32),
            pltpu.VMEM((2, _TROWS, _D), jnp.float32),
            pltpu.SemaphoreType.DMA,
            pltpu.SemaphoreType.DMA,
            pltpu.SemaphoreType.DMA,
            pltpu.SemaphoreType.DMA,
        ],
        compiler_params=_PARAMS,
    )
    def k(t4_hbm, out_hbm, in_v, out_v, si0, si1, so0, so1):
        wid = lax.axis_index("s") * _NC + lax.axis_index("c")
        sem_in = (si0, si1)
        sem_out = (so0, so1)
        iota = lax.iota(jnp.int32, 16)

        def vj0_of(s):
            return jnp.minimum(wid * (_TSTEPS * _TK) + s * _TK, _NVJ - _TK)

        idx0 = lax.shift_right_logical(iota, 3)
        c2 = iota & 7
        cbl = jnp.zeros((16,), jnp.int32)

        def fire_in(s, p):
            vj0 = vj0_of(s)
            for ti in range(2):
                pltpu.async_copy(
                    t4_hbm.at[ti, pl.ds(vj0, _TK)],
                    in_v.at[p, ti, :, :, pl.ds(0, 128)], sem_in[p])

        def wait_in(p):
            for ti in range(2):
                pltpu.make_async_copy(
                    t4_hbm.at[ti, pl.ds(0, _TK)],
                    in_v.at[p, ti, :, :, pl.ds(0, 128)], sem_in[p]).wait()

        def wait_out(p):
            pltpu.make_async_copy(out_v.at[p], out_hbm.at[pl.ds(0, _TROWS)],
                                  sem_out[p]).wait()

        fire_in(0, 0)

        def step(i, s, p):
            wait_in(p)
            @pl.when(i > 0)
            def _():
                wait_out(p)
            if p == 0:
                fire_in(s + 1, 1)
            else:
                @pl.when(i < (_TSTEPS // 2 - 1))
                def _():
                    fire_in(s + 1, 0)

            def per_k(kk, carry):
                i1 = jnp.full((16,), kk, jnp.int32)
                for bl in range(128):
                    v = plsc.load_gather(in_v.at[p], [idx0, i1, c2, cbl + bl])
                    out_v[p, kk * 128 + bl, :] = v
                return carry
            lax.fori_loop(0, _TK, per_k, 0)

            pltpu.async_copy(out_v.at[p],
                             out_hbm.at[pl.ds(vj0_of(s) * 128, _TROWS)],
                             sem_out[p])

        def pair(i, carry):
            step(i, 2 * i, 0)
            step(i, 2 * i + 1, 1)
            return carry

        lax.fori_loop(0, _TSTEPS // 2, pair, 0)
        wait_out(0)
        wait_out(1)

    return k


def _make_gather_kernel():
    mesh = plsc.VectorSubcoreMesh(**_MESH)

    @functools.partial(
        pl.kernel,
        out_type=jax.ShapeDtypeStruct((_L, _H, 2, 8, 8, 128), jnp.float32),
        mesh=mesh,
        scratch_types=[
            pltpu.VMEM((8192,), jnp.int32),
            pltpu.VMEM((2, 1024, _D), jnp.float32),
            pltpu.VMEM((2, 2, 8, 8, 129), jnp.float32),
            pltpu.VMEM((128,), jnp.int32),
            pltpu.SemaphoreType.DMA,
            pltpu.SemaphoreType.DMA,
            pltpu.SemaphoreType.DMA,
            pltpu.SemaphoreType.DMA,
            pltpu.SemaphoreType.DMA,
            pltpu.SemaphoreType.DMA,
        ],
        compiler_params=_PARAMS,
    )
    def k(ids4_hbm, tab_hbm, off_hbm, out_hbm, ids_v, rows_v, blk_v, off_v,
          sg0, sg1, ss0, ss1, si0, si1):
        wid = lax.axis_index("s") * _NC + lax.axis_index("c")
        sem_g = (sg0, sg1)
        sem_st = (ss0, ss1)
        sid = si0
        iota = lax.iota(jnp.int32, 16)
        rvec = lax.shift_right_logical(iota, 3) * 64 + (iota & 7)
        pltpu.sync_copy(off_hbm, off_v)

        def l_of_j(j):
            return jnp.minimum(wid + _NW * j, _L - 1)

        def add_offsets(m):
            h = m & 7
            base = h * 128
            off = off_v[pl.ds(h * 16, 16)]

            def per_c(c, carry):
                w = base + lax.shift_right_logical(c, 3) * 1024 + (c & 7) * 16
                ids_v[pl.ds(w, 16)] = ids_v[pl.ds(w, 16)] + off
                return carry
            lax.fori_loop(0, 64, per_c, 0)

        def fire_gathers(m, p):
            base = (m & 7) * 128
            for tj in range(8):
                pltpu.async_copy(
                    tab_hbm.at[ids_v.at[pl.ds(base + tj * 1024, 128)]],
                    rows_v.at[p, pl.ds(tj * 128, 128)],
                    sem_g[p])

        def wait_gathers(p):
            for tj in range(8):
                pltpu.make_async_copy(
                    tab_hbm.at[ids_v.at[pl.ds(tj * 1024, 128)]],
                    rows_v.at[p, pl.ds(tj * 128, 128)],
                    sem_g[p]).wait()

        def wait_store(p):
            pltpu.make_async_copy(blk_v.at[p, :, :, :, pl.ds(0, 128)],
                                  out_hbm.at[0, 0], sem_st[p]).wait()

        ti_idx = lax.shift_right_logical(iota, 3)
        ds_idx = iota & 7

        def transpose_block(p):
            def per_tj(tj, carry):
                tj_idx = jnp.full((16,), tj, jnp.int32)
                r0 = tj * 128
                for bl in range(128):
                    lrow = jnp.full((16,), r0 + bl, jnp.int32)
                    v = plsc.load_gather(rows_v.at[p], [lrow, iota])
                    plsc.store_scatter(
                        blk_v.at[p],
                        [ti_idx, tj_idx, ds_idx,
                         jnp.full((16,), bl, jnp.int32)], v)
                return carry
            lax.fori_loop(0, 8, per_tj, 0)

        def finish_block(m, q):
            wait_gathers(q)

            @pl.when(m >= 2)
            def _():
                wait_store(q)
            transpose_block(q)
            pltpu.async_copy(blk_v.at[q, :, :, :, pl.ds(0, 128)],
                             out_hbm.at[l_of_j(lax.shift_right_logical(m, 3)),
                                        m & 7],
                             sem_st[q])

        def body(u, carry):
            t = 2 * u
            j = lax.shift_right_logical(u, 2)

            @pl.when(u % 4 == 0)
            def _():
                @pl.when(u > 0)
                def _():
                    finish_block(t - 1, 1)
                pltpu.async_copy(ids4_hbm.at[l_of_j(j)], ids_v, sid).wait()

            add_offsets(t)
            fire_gathers(t, 0)

            @pl.when(u % 4 != 0)
            def _():
                finish_block(t - 1, 1)

            add_offsets(t + 1)
            fire_gathers(t + 1, 1)
            finish_block(t, 0)
            return carry

        lax.fori_loop(0, _LPW * 4, body, 0)
        finish_block(_LPW * 8 - 1, 1)
        wait_store(0)
        wait_store(1)

    return k


def kernel(input_ids, table, offsets):
    assert input_ids.shape == (_B, _L, _H) and table.shape == (_VOCAB, _D)
    ids4 = (input_ids.transpose(1, 2, 0)
            .reshape(_L, _H, 8, 128)
            .transpose(0, 2, 1, 3)
            .reshape(_L, 8192))
    tpad = jnp.pad(table, ((0, _VPAD - _VOCAB), (0, 0)))
    t4 = (tpad.T
          .reshape(2, 8, _NVJ, 128)
          .transpose(0, 2, 1, 3))
    offs = jnp.repeat(offsets, 16)
    table_lin = _make_table_kernel()(t4)
    out6 = _make_gather_kernel()(ids4, table_lin, offs)
    out = (out6.transpose(3, 5, 0, 1, 2, 4)
           .reshape(_B, _L, _H, _D))
    return out


# speedup vs baseline: 1.0067x; 1.0067x over previous
import functools

import jax
import jax.numpy as jnp
from jax import lax
from jax.experimental import pallas as pl
from jax.experimental.pallas import tpu as pltpu
from jax.experimental.pallas import tpu_sc as plsc

_B, _L, _H, _D = 1024, 200, 8, 16
_VOCAB = 800532
_VPAD = 800640
_NVJ = _VPAD // 128
_NC, _NS = 2, 16
_NW = _NC * _NS

_TK = 7
_TSTEPS = 28
_TIN = _TK * 1024
_TROWS = _TK * 128

_LPW = 7

_MESH = dict(core_axis_name="c", subcore_axis_name="s")
_PARAMS = pltpu.CompilerParams(use_tc_tiling_on_sc=False,
                               needs_layout_passes=False)


def _make_table_kernel():
    mesh = plsc.VectorSubcoreMesh(**_MESH)

    @functools.partial(
        pl.kernel,
        out_type=jax.ShapeDtypeStruct((_VPAD, _D), jnp.float32),
        mesh=mesh,
        scratch_types=[
            pltpu.VMEM((2, 2, _TK, 8, 129), jnp.float32),
            pltpu.VMEM((2, _TROWS, _D), jnp.float32),
            pltpu.SemaphoreType.DMA,
            pltpu.SemaphoreType.DMA,
            pltpu.SemaphoreType.DMA,
            pltpu.SemaphoreType.DMA,
        ],
        compiler_params=_PARAMS,
    )
    def k(t4_hbm, out_hbm, in_v, out_v, si0, si1, so0, so1):
        wid = lax.axis_index("s") * _NC + lax.axis_index("c")
        sem_in = (si0, si1)
        sem_out = (so0, so1)
        iota = lax.iota(jnp.int32, 16)

        def vj0_of(s):
            return jnp.minimum(wid * (_TSTEPS * _TK) + s * _TK, _NVJ - _TK)

        idx0 = lax.shift_right_logical(iota, 3)
        c2 = iota & 7
        cbl = jnp.zeros((16,), jnp.int32)

        def fire_in(s, p):
            vj0 = vj0_of(s)
            for ti in range(2):
                pltpu.async_copy(
                    t4_hbm.at[ti, pl.ds(vj0, _TK)],
                    in_v.at[p, ti, :, :, pl.ds(0, 128)], sem_in[p])

        def wait_in(p):
            for ti in range(2):
                pltpu.make_async_copy(
                    t4_hbm.at[ti, pl.ds(0, _TK)],
                    in_v.at[p, ti, :, :, pl.ds(0, 128)], sem_in[p]).wait()

        def wait_out(p):
            pltpu.make_async_copy(out_v.at[p], out_hbm.at[pl.ds(0, _TROWS)],
                                  sem_out[p]).wait()

        fire_in(0, 0)

        def step(i, s, p):
            wait_in(p)
            @pl.when(i > 0)
            def _():
                wait_out(p)
            if p == 0:
                fire_in(s + 1, 1)
            else:
                @pl.when(i < (_TSTEPS // 2 - 1))
                def _():
                    fire_in(s + 1, 0)

            def per_k(kk, carry):
                i1 = jnp.full((16,), kk, jnp.int32)
                for bl in range(128):
                    v = plsc.load_gather(in_v.at[p], [idx0, i1, c2, cbl + bl])
                    out_v[p, kk * 128 + bl, :] = v
                return carry
            lax.fori_loop(0, _TK, per_k, 0)

            pltpu.async_copy(out_v.at[p],
                             out_hbm.at[pl.ds(vj0_of(s) * 128, _TROWS)],
                             sem_out[p])

        def pair(i, carry):
            step(i, 2 * i, 0)
            step(i, 2 * i + 1, 1)
            return carry

        lax.fori_loop(0, _TSTEPS // 2, pair, 0)
        wait_out(0)
        wait_out(1)

    return k


def _make_gather_kernel():
    mesh = plsc.VectorSubcoreMesh(**_MESH)

    @functools.partial(
        pl.kernel,
        out_type=jax.ShapeDtypeStruct((_L, _H, 2, 8, 8, 128), jnp.float32),
        mesh=mesh,
        scratch_types=[
            pltpu.VMEM((8192,), jnp.int32),
            pltpu.VMEM((2, 1024, _D), jnp.float32),
            pltpu.VMEM((2, 2, 8, 8, 129), jnp.float32),
            pltpu.VMEM((128,), jnp.int32),
            pltpu.SemaphoreType.DMA,
            pltpu.SemaphoreType.DMA,
            pltpu.SemaphoreType.DMA,
            pltpu.SemaphoreType.DMA,
            pltpu.SemaphoreType.DMA,
            pltpu.SemaphoreType.DMA,
        ],
        compiler_params=_PARAMS,
    )
    def k(ids4_hbm, tab_hbm, off_hbm, out_hbm, ids_v, rows_v, blk_v, off_v,
          sg0, sg1, ss0, ss1, si0, si1):
        wid = lax.axis_index("s") * _NC + lax.axis_index("c")
        sem_g = (sg0, sg1)
        sem_st = (ss0, ss1)
        sid = si0
        iota = lax.iota(jnp.int32, 16)
        rvec = lax.shift_right_logical(iota, 3) * 64 + (iota & 7)
        pltpu.sync_copy(off_hbm, off_v)

        def l_of_j(j):
            return jnp.minimum(wid + _NW * j, _L - 1)

        def add_offsets(m):
            h = m & 7
            base = h * 128
            off = off_v[pl.ds(h * 16, 16)]

            def per_c(c, carry):
                w = base + lax.shift_right_logical(c, 3) * 1024 + (c & 7) * 16
                ids_v[pl.ds(w, 16)] = ids_v[pl.ds(w, 16)] + off
                return carry
            lax.fori_loop(0, 64, per_c, 0)

        def fire_gathers(m, p):
            base = (m & 7) * 128
            for tj in range(8):
                pltpu.async_copy(
                    tab_hbm.at[ids_v.at[pl.ds(base + tj * 1024, 128)]],
                    rows_v.at[p, pl.ds(tj * 128, 128)],
                    sem_g[p])

        def wait_gathers(p):
            for tj in range(8):
                pltpu.make_async_copy(
                    tab_hbm.at[ids_v.at[pl.ds(tj * 1024, 128)]],
                    rows_v.at[p, pl.ds(tj * 128, 128)],
                    sem_g[p]).wait()

        def wait_store(p):
            pltpu.make_async_copy(blk_v.at[p, :, :, :, pl.ds(0, 128)],
                                  out_hbm.at[0, 0], sem_st[p]).wait()

        ti_idx = lax.shift_right_logical(iota, 3)
        ds_idx = iota & 7

        def transpose_block(p):
            def per_tj(tj, carry):
                tj_idx = jnp.full((16,), tj, jnp.int32)
                r0 = tj * 128
                for bl in range(128):
                    v = rows_v[p, r0 + bl, :]
                    plsc.store_scatter(
                        blk_v.at[p],
                        [ti_idx, tj_idx, ds_idx,
                         jnp.full((16,), bl, jnp.int32)], v)
                return carry
            lax.fori_loop(0, 8, per_tj, 0)

        def finish_block(m, q):
            wait_gathers(q)

            @pl.when(m >= 2)
            def _():
                wait_store(q)
            transpose_block(q)
            pltpu.async_copy(blk_v.at[q, :, :, :, pl.ds(0, 128)],
                             out_hbm.at[l_of_j(lax.shift_right_logical(m, 3)),
                                        m & 7],
                             sem_st[q])

        def body(u, carry):
            t = 2 * u
            j = lax.shift_right_logical(u, 2)

            @pl.when(u % 4 == 0)
            def _():
                @pl.when(u > 0)
                def _():
                    finish_block(t - 1, 1)
                pltpu.async_copy(ids4_hbm.at[l_of_j(j)], ids_v, sid).wait()

            add_offsets(t)
            fire_gathers(t, 0)

            @pl.when(u % 4 != 0)
            def _():
                finish_block(t - 1, 1)

            add_offsets(t + 1)
            fire_gathers(t + 1, 1)
            finish_block(t, 0)
            return carry

        lax.fori_loop(0, _LPW * 4, body, 0)
        finish_block(_LPW * 8 - 1, 1)
        wait_store(0)
        wait_store(1)

    return k


def kernel(input_ids, table, offsets):
    assert input_ids.shape == (_B, _L, _H) and table.shape == (_VOCAB, _D)
    ids4 = (input_ids.transpose(1, 2, 0)
            .reshape(_L, _H, 8, 128)
            .transpose(0, 2, 1, 3)
            .reshape(_L, 8192))
    tpad = jnp.pad(table, ((0, _VPAD - _VOCAB), (0, 0)))
    t4 = (tpad.T
          .reshape(2, 8, _NVJ, 128)
          .transpose(0, 2, 1, 3))
    offs = jnp.repeat(offsets, 16)
    table_lin = _make_table_kernel()(t4)
    out6 = _make_gather_kernel()(ids4, table_lin, offs)
    out = (out6.transpose(3, 5, 0, 1, 2, 4)
           .reshape(_B, _L, _H, _D))
    return out
